# baseline (speedup 1.0000x reference)
import jax
import jax.numpy as jnp
from jax.experimental import pallas as pl

NUM_LAYERS = 3


def _gcn_norm(edge_index, edge_weight, num_nodes):
    ne = edge_index.shape[1]
    if edge_weight is None:
        edge_weight = jnp.ones((ne,), dtype=jnp.float32)
    loop = jnp.arange(num_nodes, dtype=edge_index.dtype)
    row = jnp.concatenate([edge_index[0], loop])
    col = jnp.concatenate([edge_index[1], loop])
    w = jnp.concatenate([edge_weight, jnp.ones((num_nodes,), dtype=jnp.float32)])
    deg = jax.ops.segment_sum(w, col, num_segments=num_nodes)
    dinv = deg ** -0.5
    dinv = jnp.where(jnp.isfinite(dinv), dinv, 0.0)
    norm = dinv[row] * w * dinv[col]
    return row, col, norm


def _dgconv(x, edge_index, edge_weight):
    n = x.shape[0]
    row, col, norm = _gcn_norm(edge_index, edge_weight, n)
    msgs = norm[:, None] * x[row]
    return jax.ops.segment_sum(msgs, col, num_segments=n)


def _identity_pallas(x):
    def body(x_ref, o_ref):
        o_ref[...] = x_ref[...]
    return pl.pallas_call(
        body, out_shape=jax.ShapeDtypeStruct(x.shape, x.dtype))(x)


def kernel(x, edge_index, edge_in, edge_out, in_w, out_w, W0, W1, W2, b0, b1, b2, Wc, bc):
    Ws = [W0, W1, W2]
    bs = [b0, b1, b2]
    h = x
    for i in range(NUM_LAYERS):
        h = h @ Ws[i].T
        h1 = _dgconv(h, edge_index, None) + bs[i]
        h2 = _dgconv(h, edge_in, in_w) + bs[i]
        h3 = _dgconv(h, edge_out, out_w) + bs[i]
        h = jax.nn.relu(jnp.concatenate([h1, h2, h3], axis=-1))
    h = _identity_pallas(h)
    y = h @ Wc.T + bc
    return (h, jax.nn.log_softmax(y, axis=1))


# speedup vs baseline: 5.3826x; 5.3826x over previous
import functools

import jax
import jax.numpy as jnp
from jax import lax
from jax.experimental import pallas as pl
from jax.experimental.pallas import tpu as pltpu
from jax.experimental.pallas import tpu_sc as plsc

_N = 10000
_PN = 10240
_E = 320000
_D = 128
_OUT = 40
_C = 80
_CH = _E // 32 // _C
_NC = 2
_NS = 16
_RPT = _PN // _NS
_R = 400
_G = _N // _R
_DH = _D // 2

_f32 = jnp.float32


def _sc_degrees(col0, col1, col2, w16_1, w16_2):
    mesh = plsc.VectorSubcoreMesh(core_axis_name="c", subcore_axis_name="s")

    @functools.partial(
        pl.kernel,
        out_type=jax.ShapeDtypeStruct((_NC * 3 * _PN, 16), _f32),
        mesh=mesh,
        compiler_params=pltpu.CompilerParams(use_tc_tiling_on_sc=False),
        scratch_types=[
            pltpu.VMEM((_CH, _C), jnp.int32),
            pltpu.VMEM((_C, 16), _f32),
            pltpu.VMEM((128, 16), _f32),
            pltpu.VMEM((128, 16), _f32),
            pltpu.VMEM_SHARED((_PN, 16), _f32),
        ],
    )
    def k(col0_h, col1_h, col2_h, w1_h, w2_h, out_h,
          colbuf, wrow, zb, bounce, acc):
        c = lax.axis_index("c")
        s = lax.axis_index("s")
        wid = c * _NS + s

        def zrow(r, carry):
            zb[r, :] = jnp.zeros((16,), _f32)
            return carry
        lax.fori_loop(0, 128, zrow, 0)

        cols_h = [col0_h, col1_h, col2_h]
        ws_h = [None, w1_h, w2_h]
        for kk in range(3):
            for z in range(5):
                pltpu.sync_copy(zb, acc.at[pl.ds(s * _RPT + z * 128, 128)])
            plsc.subcore_barrier()
            pltpu.sync_copy(cols_h[kk].at[wid], colbuf)
            if ws_h[kk] is None:
                def orow(r, carry):
                    wrow[r, :] = jnp.full((16,), 1.0, _f32)
                    return carry
                lax.fori_loop(0, _C, orow, 0)

                def chunk0(j, carry):
                    pltpu.sync_copy(wrow, acc.at[colbuf.at[j]], add=True)
                    return carry
                lax.fori_loop(0, _CH, chunk0, 0)
            else:
                w_h = ws_h[kk]

                def chunkw(j, carry):
                    base = pl.multiple_of((wid * _CH + j) * _C, 8)
                    pltpu.sync_copy(w_h.at[pl.ds(base, _C)], wrow)
                    pltpu.sync_copy(wrow, acc.at[colbuf.at[j]], add=True)
                    return carry
                lax.fori_loop(0, _CH, chunkw, 0)
            plsc.subcore_barrier()
            for z in range(5):
                pltpu.sync_copy(acc.at[pl.ds(s * _RPT + z * 128, 128)], bounce)
                pltpu.sync_copy(
                    bounce,
                    out_h.at[pl.ds((c * 3 + kk) * _PN + s * _RPT + z * 128,
                                   128)])

    return k(col0, col1, col2, w16_1, w16_2)


def _sc_propagate(g0, g1, g2, r0, c0, r1, c1, w1, r2, c2, w2):
    mesh = plsc.VectorSubcoreMesh(core_axis_name="c", subcore_axis_name="s")

    @functools.partial(
        pl.kernel,
        out_type=[jax.ShapeDtypeStruct((_NC * _PN, _DH), _f32)] * 3,
        mesh=mesh,
        compiler_params=pltpu.CompilerParams(use_tc_tiling_on_sc=False),
        scratch_types=[
            pltpu.VMEM((2 * _CH, _C), jnp.int32),
            pltpu.VMEM((2 * _CH, _C), jnp.int32),
            pltpu.VMEM((_C, 16), _f32),
            pltpu.VMEM((_C, _DH), _f32),
            pltpu.VMEM((128, _DH), _f32),
            pltpu.VMEM((128, _DH), _f32),
            pltpu.VMEM_SHARED((_PN, _DH), _f32),
            pltpu.SemaphoreType.DMA,
        ],
    )
    def k(g0_h, g1_h, g2_h, r0_h, c0_h, r1_h, c1_h, w1_h, r2_h, c2_h, w2_h,
          o0_h, o1_h, o2_h,
          rowbuf, colbuf, wrow, rv, zr, bounce, acc, gsem):
        c = lax.axis_index("c")
        s = lax.axis_index("s")
        goff = c * _N

        def zrow(r, carry):
            for t in range(_DH // 16):
                zr[r, pl.ds(t * 16, 16)] = jnp.zeros((16,), _f32)
            return carry
        lax.fori_loop(0, 128, zrow, 0)

        gs_h = [g0_h, g1_h, g2_h]
        rows_h = [r0_h, r1_h, r2_h]
        cols_h = [c0_h, c1_h, c2_h]
        ws_h = [None, w1_h, w2_h]
        outs_h = [o0_h, o1_h, o2_h]
        for kk in range(3):
            for z in range(5):
                pltpu.sync_copy(zr, acc.at[pl.ds(s * _RPT + z * 128, 128)])
            plsc.subcore_barrier()
            for half in range(2):
                pltpu.sync_copy(rows_h[kk].at[2 * s + half],
                                rowbuf.at[pl.ds(half * _CH, _CH)])
                pltpu.sync_copy(cols_h[kk].at[2 * s + half],
                                colbuf.at[pl.ds(half * _CH, _CH)])

            def adj(j, carry):
                for rb in range(_C // 16):
                    rowbuf[j, pl.ds(rb * 16, 16)] = (
                        rowbuf[j, pl.ds(rb * 16, 16)] + goff)
                return carry
            lax.fori_loop(0, 2 * _CH, adj, 0)

            if ws_h[kk] is None:
                def chunk0(j, carry):
                    pltpu.async_copy(gs_h[0].at[rowbuf.at[j]], rv, gsem).wait()
                    pltpu.sync_copy(rv, acc.at[colbuf.at[j]], add=True)
                    return carry
                lax.fori_loop(0, 2 * _CH, chunk0, 0)
            else:
                g_h = gs_h[kk]
                w_h = ws_h[kk]

                def chunkw(j, carry):
                    pltpu.async_copy(g_h.at[rowbuf.at[j]], rv, gsem).wait()
                    base = pl.multiple_of((2 * s * _CH + j) * _C, 8)
                    pltpu.sync_copy(w_h.at[pl.ds(base, _C)], wrow)

                    def scale(r, carry2):
                        wv = wrow[r, :]
                        for t in range(_DH // 16):
                            rv[r, pl.ds(t * 16, 16)] = (
                                rv[r, pl.ds(t * 16, 16)] * wv)
                        return carry2
                    lax.fori_loop(0, _C, scale, 0)
                    pltpu.sync_copy(rv, acc.at[colbuf.at[j]], add=True)
                    return carry
                lax.fori_loop(0, 2 * _CH, chunkw, 0)
            plsc.subcore_barrier()
            for z in range(5):
                pltpu.sync_copy(acc.at[pl.ds(s * _RPT + z * 128, 128)], bounce)
                pltpu.sync_copy(
                    bounce,
                    outs_h[kk].at[pl.ds(c * _PN + s * _RPT + z * 128, 128)])

    return k(g0, g1, g2, r0, c0, r1, c1, w1, r2, c2, w2)


def _dinvs(dp, kk):
    deg = 1.0 + dp[0, kk, :, 0:1] + dp[1, kk, :, 0:1]
    dinv = lax.rsqrt(deg)
    return dinv, 1.0 / deg


def _store_g(g_ref, ga):
    g_ref[0] = ga[:, :_DH]
    g_ref[1] = ga[:, _DH:]


def _tc_layer0(x, W0, degp):
    def body(x_ref, w_ref, dp_ref, hw_ref, g0_ref, g1_ref, g2_ref):
        xa = x_ref[...]
        hw = lax.dot_general(xa, w_ref[...], (((1,), (1,)), ((), ())),
                             preferred_element_type=_f32)
        hw_ref[...] = hw
        dp = dp_ref[...]
        for kk, g_ref in enumerate((g0_ref, g1_ref, g2_ref)):
            dinv, _ = _dinvs(dp, kk)
            _store_g(g_ref, dinv * hw)

    return pl.pallas_call(
        body,
        grid=(_G,),
        in_specs=[
            pl.BlockSpec((_R, _D), lambda i: (i, 0)),
            pl.BlockSpec((_D, _D), lambda i: (0, 0)),
            pl.BlockSpec((2, 3, _R, 16), lambda i: (0, 0, i, 0)),
        ],
        out_specs=[pl.BlockSpec((_R, _D), lambda i: (i, 0))]
        + [pl.BlockSpec((2, _R, _DH), lambda i: (0, i, 0))] * 3,
        out_shape=[jax.ShapeDtypeStruct((_N, _D), _f32)]
        + [jax.ShapeDtypeStruct((2, _N, _DH), _f32)] * 3,
    )(x, W0, degp)


def _tc_combine(p0, p1, p2, hw, degp, b, Wn):
    def body(p0_ref, p1_ref, p2_ref, hw_ref, dp_ref, b_ref, wn_ref,
             hwn_ref, g0_ref, g1_ref, g2_ref):
        hw_ = hw_ref[...]
        dp = dp_ref[...]
        bv = b_ref[...]
        wn = wn_ref[...]
        acc = jnp.zeros((_R, _D), _f32)
        for kk, p_ref in enumerate((p0_ref, p1_ref, p2_ref)):
            p = p_ref[...]
            seg = jnp.concatenate([p[0], p[1]], axis=1)
            dinv, dinv2 = _dinvs(dp, kk)
            sk = dinv * seg + dinv2 * hw_ + bv
            rk = jnp.maximum(sk, 0.0)
            acc = acc + lax.dot_general(
                rk, wn[:, kk * _D:(kk + 1) * _D], (((1,), (1,)), ((), ())),
                preferred_element_type=_f32)
        hwn_ref[...] = acc
        for kk, g_ref in enumerate((g0_ref, g1_ref, g2_ref)):
            dinv, _ = _dinvs(dp, kk)
            _store_g(g_ref, dinv * acc)

    return pl.pallas_call(
        body,
        grid=(_G,),
        in_specs=[
            pl.BlockSpec((2, _R, _DH), lambda i: (0, i, 0)),
            pl.BlockSpec((2, _R, _DH), lambda i: (0, i, 0)),
            pl.BlockSpec((2, _R, _DH), lambda i: (0, i, 0)),
            pl.BlockSpec((_R, _D), lambda i: (i, 0)),
            pl.BlockSpec((2, 3, _R, 16), lambda i: (0, 0, i, 0)),
            pl.BlockSpec((1, _D), lambda i: (0, 0)),
            pl.BlockSpec((_D, 3 * _D), lambda i: (0, 0)),
        ],
        out_specs=[pl.BlockSpec((_R, _D), lambda i: (i, 0))]
        + [pl.BlockSpec((2, _R, _DH), lambda i: (0, i, 0))] * 3,
        out_shape=[jax.ShapeDtypeStruct((_N, _D), _f32)]
        + [jax.ShapeDtypeStruct((2, _N, _DH), _f32)] * 3,
    )(p0, p1, p2, hw, degp, b, Wn)


def _tc_final(p0, p1, p2, hw, degp, b, Wc, bc):
    def body(p0_ref, p1_ref, p2_ref, hw_ref, dp_ref, b_ref, wc_ref, bc_ref,
             h_ref, ls_ref):
        hw_ = hw_ref[...]
        dp = dp_ref[...]
        bv = b_ref[...]
        wc = wc_ref[...]
        y = jnp.zeros((_R, _OUT), _f32) + bc_ref[...]
        for kk, p_ref in enumerate((p0_ref, p1_ref, p2_ref)):
            p = p_ref[...]
            seg = jnp.concatenate([p[0], p[1]], axis=1)
            dinv, dinv2 = _dinvs(dp, kk)
            sk = dinv * seg + dinv2 * hw_ + bv
            rk = jnp.maximum(sk, 0.0)
            h_ref[:, kk * _D:(kk + 1) * _D] = rk
            y = y + lax.dot_general(
                rk, wc[:, kk * _D:(kk + 1) * _D], (((1,), (1,)), ((), ())),
                preferred_element_type=_f32)
        m = jnp.max(y, axis=1, keepdims=True)
        z = y - m
        ls_ref[...] = z - jnp.log(jnp.sum(jnp.exp(z), axis=1, keepdims=True))

    return pl.pallas_call(
        body,
        grid=(_G,),
        in_specs=[
            pl.BlockSpec((2, _R, _DH), lambda i: (0, i, 0)),
            pl.BlockSpec((2, _R, _DH), lambda i: (0, i, 0)),
            pl.BlockSpec((2, _R, _DH), lambda i: (0, i, 0)),
            pl.BlockSpec((_R, _D), lambda i: (i, 0)),
            pl.BlockSpec((2, 3, _R, 16), lambda i: (0, 0, i, 0)),
            pl.BlockSpec((1, _D), lambda i: (0, 0)),
            pl.BlockSpec((_OUT, 3 * _D), lambda i: (0, 0)),
            pl.BlockSpec((1, _OUT), lambda i: (0, 0)),
        ],
        out_specs=[
            pl.BlockSpec((_R, 3 * _D), lambda i: (i, 0)),
            pl.BlockSpec((_R, _OUT), lambda i: (i, 0)),
        ],
        out_shape=[
            jax.ShapeDtypeStruct((_N, 3 * _D), _f32),
            jax.ShapeDtypeStruct((_N, _OUT), _f32),
        ],
    )(p0, p1, p2, hw, degp, b, Wc, bc)


def kernel(x, edge_index, edge_in, edge_out, in_w, out_w,
           W0, W1, W2, b0, b1, b2, Wc, bc):
    shp = (32, _CH, _C)
    r0 = edge_index[0].reshape(shp)
    c0 = edge_index[1].reshape(shp)
    r1 = edge_in[0].reshape(shp)
    c1 = edge_in[1].reshape(shp)
    r2 = edge_out[0].reshape(shp)
    c2 = edge_out[1].reshape(shp)
    w1 = jnp.broadcast_to(in_w[:, None], (_E, 16))
    w2 = jnp.broadcast_to(out_w[:, None], (_E, 16))

    degp = _sc_degrees(c0, c1, c2, w1, w2)
    degp = degp.reshape(_NC, 3, _PN, 16)[:, :, :_N, :]

    hw, g0, g1, g2 = _tc_layer0(x, W0, degp)
    Wn = [W1, W2]
    bs = [b0, b1, b2]
    for layer in range(3):
        p0, p1, p2 = _sc_propagate(
            g0.reshape(2 * _N, _DH), g1.reshape(2 * _N, _DH),
            g2.reshape(2 * _N, _DH), r0, c0, r1, c1, w1, r2, c2, w2)
        p0 = p0.reshape(_NC, _PN, _DH)[:, :_N, :]
        p1 = p1.reshape(_NC, _PN, _DH)[:, :_N, :]
        p2 = p2.reshape(_NC, _PN, _DH)[:, :_N, :]
        if layer < 2:
            hw, g0, g1, g2 = _tc_combine(p0, p1, p2, hw, degp, bs[layer],
                                         Wn[layer])
        else:
            h, ls = _tc_final(p0, p1, p2, hw, degp, bs[layer], Wc,
                              bc.reshape(1, _OUT))
    return (h, ls)


# speedup vs baseline: 9.7590x; 1.8131x over previous
import functools

import jax
import jax.numpy as jnp
from jax import lax
from jax.experimental import pallas as pl
from jax.experimental.pallas import tpu as pltpu
from jax.experimental.pallas import tpu_sc as plsc

_N = 10000
_PN = 10240
_E = 320000
_D = 128
_OUT = 40
_C = 80
_CH = _E // 32 // _C
_NC = 2
_NS = 16
_RPT = _PN // _NS
_R = 400
_G = _N // _R
_DH = _D // 2

_f32 = jnp.float32


def _sc_degrees(col0, col1, col2, w16_1, w16_2):
    mesh = plsc.VectorSubcoreMesh(core_axis_name="c", subcore_axis_name="s")

    @functools.partial(
        pl.kernel,
        out_type=jax.ShapeDtypeStruct((_NC * 3 * _PN, 16), _f32),
        mesh=mesh,
        compiler_params=pltpu.CompilerParams(use_tc_tiling_on_sc=False),
        scratch_types=[
            pltpu.VMEM((_CH, _C), jnp.int32),
            pltpu.VMEM((_C, 16), _f32),
            pltpu.VMEM((128, 16), _f32),
            pltpu.VMEM((128, 16), _f32),
            pltpu.VMEM_SHARED((_PN, 16), _f32),
        ],
    )
    def k(col0_h, col1_h, col2_h, w1_h, w2_h, out_h,
          colbuf, wrow, zb, bounce, acc):
        c = lax.axis_index("c")
        s = lax.axis_index("s")
        wid = c * _NS + s

        def zrow(r, carry):
            zb[r, :] = jnp.zeros((16,), _f32)
            return carry
        lax.fori_loop(0, 128, zrow, 0)

        cols_h = [col0_h, col1_h, col2_h]
        ws_h = [None, w1_h, w2_h]
        for kk in range(3):
            for z in range(5):
                pltpu.sync_copy(zb, acc.at[pl.ds(s * _RPT + z * 128, 128)])
            plsc.subcore_barrier()
            pltpu.sync_copy(cols_h[kk].at[wid], colbuf)
            if ws_h[kk] is None:
                def orow(r, carry):
                    wrow[r, :] = jnp.full((16,), 1.0, _f32)
                    return carry
                lax.fori_loop(0, _C, orow, 0)

                def chunk0(j, carry):
                    pltpu.sync_copy(wrow, acc.at[colbuf.at[j]], add=True)
                    return carry
                lax.fori_loop(0, _CH, chunk0, 0)
            else:
                w_h = ws_h[kk]

                def chunkw(j, carry):
                    base = pl.multiple_of((wid * _CH + j) * _C, 8)
                    pltpu.sync_copy(w_h.at[pl.ds(base, _C)], wrow)
                    pltpu.sync_copy(wrow, acc.at[colbuf.at[j]], add=True)
                    return carry
                lax.fori_loop(0, _CH, chunkw, 0)
            plsc.subcore_barrier()
            for z in range(5):
                pltpu.sync_copy(acc.at[pl.ds(s * _RPT + z * 128, 128)], bounce)
                pltpu.sync_copy(
                    bounce,
                    out_h.at[pl.ds((c * 3 + kk) * _PN + s * _RPT + z * 128,
                                   128)])

    return k(col0, col1, col2, w16_1, w16_2)


def _sc_propagate(g0, g1, g2, r0, c0, r1, c1, w1, r2, c2, w2):
    mesh = plsc.VectorSubcoreMesh(core_axis_name="c", subcore_axis_name="s")

    @functools.partial(
        pl.kernel,
        out_type=[jax.ShapeDtypeStruct((_NC * _PN, _DH), _f32)] * 3,
        mesh=mesh,
        compiler_params=pltpu.CompilerParams(use_tc_tiling_on_sc=False),
        scratch_types=[
            pltpu.VMEM((2 * _CH, _C), jnp.int32),
            pltpu.VMEM((2 * _CH, _C), jnp.int32),
            pltpu.VMEM((_C, 16), _f32),
            pltpu.VMEM((_C, 16), _f32),
            pltpu.VMEM((_C, _DH), _f32),
            pltpu.VMEM((_C, _DH), _f32),
            pltpu.VMEM((128, _DH), _f32),
            pltpu.VMEM((128, _DH), _f32),
            pltpu.VMEM_SHARED((_PN, _DH), _f32),
            pltpu.SemaphoreType.DMA,
            pltpu.SemaphoreType.DMA,
            pltpu.SemaphoreType.DMA,
            pltpu.SemaphoreType.DMA,
        ],
    )
    def k(g0_h, g1_h, g2_h, r0_h, c0_h, r1_h, c1_h, w1_h, r2_h, c2_h, w2_h,
          o0_h, o1_h, o2_h,
          rowbuf, colbuf, wrA, wrB, rvA, rvB, zr, bounce, acc,
          gsA, gsB, ssA, ssB):
        c = lax.axis_index("c")
        s = lax.axis_index("s")
        goff = c * _N
        nj = 2 * _CH

        def zrow(r, carry):
            for t in range(_DH // 16):
                zr[r, pl.ds(t * 16, 16)] = jnp.zeros((16,), _f32)
            return carry
        lax.fori_loop(0, 128, zrow, 0)

        gs_h = [g0_h, g1_h, g2_h]
        rows_h = [r0_h, r1_h, r2_h]
        cols_h = [c0_h, c1_h, c2_h]
        ws_h = [None, w1_h, w2_h]
        outs_h = [o0_h, o1_h, o2_h]
        bufs = [(rvA, wrA, gsA, ssA), (rvB, wrB, gsB, ssB)]
        for kk in range(3):
            for z in range(5):
                pltpu.sync_copy(zr, acc.at[pl.ds(s * _RPT + z * 128, 128)])
            plsc.subcore_barrier()
            for half in range(2):
                pltpu.sync_copy(rows_h[kk].at[2 * s + half],
                                rowbuf.at[pl.ds(half * _CH, _CH)])
                pltpu.sync_copy(cols_h[kk].at[2 * s + half],
                                colbuf.at[pl.ds(half * _CH, _CH)])

            def adj(j, carry):
                for rb in range(_C // 16):
                    rowbuf[j, pl.ds(rb * 16, 16)] = (
                        rowbuf[j, pl.ds(rb * 16, 16)] + goff)
                return carry
            lax.fori_loop(0, 2 * _CH, adj, 0)

            g_h = gs_h[kk]
            w_h = ws_h[kk]
            weighted = w_h is not None

            def issue(j, buf):
                rvb, wrb, gsb, _ = bufs[buf]
                pltpu.async_copy(g_h.at[rowbuf.at[j]], rvb, gsb)
                if weighted:
                    base = pl.multiple_of((2 * s * _CH + j) * _C, 8)
                    pltpu.async_copy(w_h.at[pl.ds(base, _C)], wrb, gsb)

            def wait_gather(buf):
                rvb, wrb, gsb, _ = bufs[buf]
                pltpu.make_async_copy(g_h.at[rowbuf.at[0]], rvb, gsb).wait()
                if weighted:
                    pltpu.make_async_copy(
                        w_h.at[pl.ds(0, _C)], wrb, gsb).wait()

            def wait_scatter(buf):
                rvb, _, _, ssb = bufs[buf]
                pltpu.make_async_copy(rvb, acc.at[colbuf.at[0]], ssb).wait()

            issue(0, 0)

            def pair(i, carry):
                for b in range(2):
                    j = 2 * i + b
                    rvb, wrb, _, ssb = bufs[b]

                    @pl.when(j + 1 < nj)
                    def _():
                        @pl.when(j >= 1)
                        def _():
                            wait_scatter(1 - b)
                        issue(j + 1, 1 - b)

                    wait_gather(b)
                    if weighted:
                        def scale(r, carry2):
                            wv = wrb[r, :]
                            for t in range(_DH // 16):
                                rvb[r, pl.ds(t * 16, 16)] = (
                                    rvb[r, pl.ds(t * 16, 16)] * wv)
                            return carry2
                        lax.fori_loop(0, _C, scale, 0)
                    pltpu.async_copy(rvb, acc.at[colbuf.at[j]], ssb,
                                     add=True)
                return carry
            lax.fori_loop(0, _CH, pair, 0)
            wait_scatter(0)
            wait_scatter(1)
            plsc.subcore_barrier()
            for z in range(5):
                pltpu.sync_copy(acc.at[pl.ds(s * _RPT + z * 128, 128)], bounce)
                pltpu.sync_copy(
                    bounce,
                    outs_h[kk].at[pl.ds(c * _PN + s * _RPT + z * 128, 128)])

    return k(g0, g1, g2, r0, c0, r1, c1, w1, r2, c2, w2)


def _dinvs(dp, kk):
    deg = 1.0 + dp[0, kk, :, 0:1] + dp[1, kk, :, 0:1]
    dinv = lax.rsqrt(deg)
    return dinv, 1.0 / deg


def _store_g(g_ref, ga):
    g_ref[0] = ga[:, :_DH]
    g_ref[1] = ga[:, _DH:]


def _tc_layer0(x, W0, degp):
    def body(x_ref, w_ref, dp_ref, hw_ref, g0_ref, g1_ref, g2_ref):
        xa = x_ref[...]
        hw = lax.dot_general(xa, w_ref[...], (((1,), (1,)), ((), ())),
                             preferred_element_type=_f32)
        hw_ref[...] = hw
        dp = dp_ref[...]
        for kk, g_ref in enumerate((g0_ref, g1_ref, g2_ref)):
            dinv, _ = _dinvs(dp, kk)
            _store_g(g_ref, dinv * hw)

    return pl.pallas_call(
        body,
        grid=(_G,),
        in_specs=[
            pl.BlockSpec((_R, _D), lambda i: (i, 0)),
            pl.BlockSpec((_D, _D), lambda i: (0, 0)),
            pl.BlockSpec((2, 3, _R, 16), lambda i: (0, 0, i, 0)),
        ],
        out_specs=[pl.BlockSpec((_R, _D), lambda i: (i, 0))]
        + [pl.BlockSpec((2, _R, _DH), lambda i: (0, i, 0))] * 3,
        out_shape=[jax.ShapeDtypeStruct((_N, _D), _f32)]
        + [jax.ShapeDtypeStruct((2, _N, _DH), _f32)] * 3,
    )(x, W0, degp)


def _tc_combine(p0, p1, p2, hw, degp, b, Wn):
    def body(p0_ref, p1_ref, p2_ref, hw_ref, dp_ref, b_ref, wn_ref,
             hwn_ref, g0_ref, g1_ref, g2_ref):
        hw_ = hw_ref[...]
        dp = dp_ref[...]
        bv = b_ref[...]
        wn = wn_ref[...]
        acc = jnp.zeros((_R, _D), _f32)
        for kk, p_ref in enumerate((p0_ref, p1_ref, p2_ref)):
            p = p_ref[...]
            seg = jnp.concatenate([p[0], p[1]], axis=1)
            dinv, dinv2 = _dinvs(dp, kk)
            sk = dinv * seg + dinv2 * hw_ + bv
            rk = jnp.maximum(sk, 0.0)
            acc = acc + lax.dot_general(
                rk, wn[:, kk * _D:(kk + 1) * _D], (((1,), (1,)), ((), ())),
                preferred_element_type=_f32)
        hwn_ref[...] = acc
        for kk, g_ref in enumerate((g0_ref, g1_ref, g2_ref)):
            dinv, _ = _dinvs(dp, kk)
            _store_g(g_ref, dinv * acc)

    return pl.pallas_call(
        body,
        grid=(_G,),
        in_specs=[
            pl.BlockSpec((2, _R, _DH), lambda i: (0, i, 0)),
            pl.BlockSpec((2, _R, _DH), lambda i: (0, i, 0)),
            pl.BlockSpec((2, _R, _DH), lambda i: (0, i, 0)),
            pl.BlockSpec((_R, _D), lambda i: (i, 0)),
            pl.BlockSpec((2, 3, _R, 16), lambda i: (0, 0, i, 0)),
            pl.BlockSpec((1, _D), lambda i: (0, 0)),
            pl.BlockSpec((_D, 3 * _D), lambda i: (0, 0)),
        ],
        out_specs=[pl.BlockSpec((_R, _D), lambda i: (i, 0))]
        + [pl.BlockSpec((2, _R, _DH), lambda i: (0, i, 0))] * 3,
        out_shape=[jax.ShapeDtypeStruct((_N, _D), _f32)]
        + [jax.ShapeDtypeStruct((2, _N, _DH), _f32)] * 3,
    )(p0, p1, p2, hw, degp, b, Wn)


def _tc_final(p0, p1, p2, hw, degp, b, Wc, bc):
    def body(p0_ref, p1_ref, p2_ref, hw_ref, dp_ref, b_ref, wc_ref, bc_ref,
             h_ref, ls_ref):
        hw_ = hw_ref[...]
        dp = dp_ref[...]
        bv = b_ref[...]
        wc = wc_ref[...]
        y = jnp.zeros((_R, _OUT), _f32) + bc_ref[...]
        for kk, p_ref in enumerate((p0_ref, p1_ref, p2_ref)):
            p = p_ref[...]
            seg = jnp.concatenate([p[0], p[1]], axis=1)
            dinv, dinv2 = _dinvs(dp, kk)
            sk = dinv * seg + dinv2 * hw_ + bv
            rk = jnp.maximum(sk, 0.0)
            h_ref[:, kk * _D:(kk + 1) * _D] = rk
            y = y + lax.dot_general(
                rk, wc[:, kk * _D:(kk + 1) * _D], (((1,), (1,)), ((), ())),
                preferred_element_type=_f32)
        m = jnp.max(y, axis=1, keepdims=True)
        z = y - m
        ls_ref[...] = z - jnp.log(jnp.sum(jnp.exp(z), axis=1, keepdims=True))

    return pl.pallas_call(
        body,
        grid=(_G,),
        in_specs=[
            pl.BlockSpec((2, _R, _DH), lambda i: (0, i, 0)),
            pl.BlockSpec((2, _R, _DH), lambda i: (0, i, 0)),
            pl.BlockSpec((2, _R, _DH), lambda i: (0, i, 0)),
            pl.BlockSpec((_R, _D), lambda i: (i, 0)),
            pl.BlockSpec((2, 3, _R, 16), lambda i: (0, 0, i, 0)),
            pl.BlockSpec((1, _D), lambda i: (0, 0)),
            pl.BlockSpec((_OUT, 3 * _D), lambda i: (0, 0)),
            pl.BlockSpec((1, _OUT), lambda i: (0, 0)),
        ],
        out_specs=[
            pl.BlockSpec((_R, 3 * _D), lambda i: (i, 0)),
            pl.BlockSpec((_R, _OUT), lambda i: (i, 0)),
        ],
        out_shape=[
            jax.ShapeDtypeStruct((_N, 3 * _D), _f32),
            jax.ShapeDtypeStruct((_N, _OUT), _f32),
        ],
    )(p0, p1, p2, hw, degp, b, Wc, bc)


def kernel(x, edge_index, edge_in, edge_out, in_w, out_w,
           W0, W1, W2, b0, b1, b2, Wc, bc):
    shp = (32, _CH, _C)
    r0 = edge_index[0].reshape(shp)
    c0 = edge_index[1].reshape(shp)
    r1 = edge_in[0].reshape(shp)
    c1 = edge_in[1].reshape(shp)
    r2 = edge_out[0].reshape(shp)
    c2 = edge_out[1].reshape(shp)
    w1 = jnp.broadcast_to(in_w[:, None], (_E, 16))
    w2 = jnp.broadcast_to(out_w[:, None], (_E, 16))

    degp = _sc_degrees(c0, c1, c2, w1, w2)
    degp = degp.reshape(_NC, 3, _PN, 16)[:, :, :_N, :]

    hw, g0, g1, g2 = _tc_layer0(x, W0, degp)
    Wn = [W1, W2]
    bs = [b0, b1, b2]
    for layer in range(3):
        p0, p1, p2 = _sc_propagate(
            g0.reshape(2 * _N, _DH), g1.reshape(2 * _N, _DH),
            g2.reshape(2 * _N, _DH), r0, c0, r1, c1, w1, r2, c2, w2)
        p0 = p0.reshape(_NC, _PN, _DH)[:, :_N, :]
        p1 = p1.reshape(_NC, _PN, _DH)[:, :_N, :]
        p2 = p2.reshape(_NC, _PN, _DH)[:, :_N, :]
        if layer < 2:
            hw, g0, g1, g2 = _tc_combine(p0, p1, p2, hw, degp, bs[layer],
                                         Wn[layer])
        else:
            h, ls = _tc_final(p0, p1, p2, hw, degp, bs[layer], Wc,
                              bc.reshape(1, _OUT))
    return (h, ls)


# speedup vs baseline: 10.7588x; 1.1024x over previous
import functools

import jax
import jax.numpy as jnp
from jax import lax
from jax.experimental import pallas as pl
from jax.experimental.pallas import tpu as pltpu
from jax.experimental.pallas import tpu_sc as plsc

_N = 10000
_PN = 10240
_E = 320000
_D = 128
_OUT = 40
_C = 80
_CH = _E // 32 // _C
_NC = 2
_NS = 16
_RPT = _PN // _NS
_R = 400
_G = _N // _R
_DH = _D // 2

_f32 = jnp.float32


def _sc_degrees(col0, col1, col2, w16_1, w16_2):
    mesh = plsc.VectorSubcoreMesh(core_axis_name="c", subcore_axis_name="s")

    @functools.partial(
        pl.kernel,
        out_type=jax.ShapeDtypeStruct((_NC * 3 * _PN, 16), _f32),
        mesh=mesh,
        compiler_params=pltpu.CompilerParams(use_tc_tiling_on_sc=False),
        scratch_types=[
            pltpu.VMEM((_CH, _C), jnp.int32),
            pltpu.VMEM((_C, 16), _f32),
            pltpu.VMEM((128, 16), _f32),
            pltpu.VMEM((128, 16), _f32),
            pltpu.VMEM_SHARED((_PN, 16), _f32),
        ],
    )
    def k(col0_h, col1_h, col2_h, w1_h, w2_h, out_h,
          colbuf, wrow, zb, bounce, acc):
        c = lax.axis_index("c")
        s = lax.axis_index("s")
        wid = c * _NS + s

        def zrow(r, carry):
            zb[r, :] = jnp.zeros((16,), _f32)
            return carry
        lax.fori_loop(0, 128, zrow, 0)

        cols_h = [col0_h, col1_h, col2_h]
        ws_h = [None, w1_h, w2_h]
        for kk in range(3):
            for z in range(5):
                pltpu.sync_copy(zb, acc.at[pl.ds(s * _RPT + z * 128, 128)])
            plsc.subcore_barrier()
            pltpu.sync_copy(cols_h[kk].at[wid], colbuf)
            if ws_h[kk] is None:
                def orow(r, carry):
                    wrow[r, :] = jnp.full((16,), 1.0, _f32)
                    return carry
                lax.fori_loop(0, _C, orow, 0)

                def chunk0(j, carry):
                    pltpu.sync_copy(wrow, acc.at[colbuf.at[j]], add=True)
                    return carry
                lax.fori_loop(0, _CH, chunk0, 0)
            else:
                w_h = ws_h[kk]

                def chunkw(j, carry):
                    base = pl.multiple_of((wid * _CH + j) * _C, 8)
                    pltpu.sync_copy(w_h.at[pl.ds(base, _C)], wrow)
                    pltpu.sync_copy(wrow, acc.at[colbuf.at[j]], add=True)
                    return carry
                lax.fori_loop(0, _CH, chunkw, 0)
            plsc.subcore_barrier()
            for z in range(5):
                pltpu.sync_copy(acc.at[pl.ds(s * _RPT + z * 128, 128)], bounce)
                pltpu.sync_copy(
                    bounce,
                    out_h.at[pl.ds((c * 3 + kk) * _PN + s * _RPT + z * 128,
                                   128)])

    return k(col0, col1, col2, w16_1, w16_2)


def _sc_propagate(g0, g1, g2, r0, c0, r1, c1, w1, r2, c2, w2):
    mesh = plsc.VectorSubcoreMesh(core_axis_name="c", subcore_axis_name="s")

    @functools.partial(
        pl.kernel,
        out_type=[jax.ShapeDtypeStruct((_NC * _PN, _DH), _f32)] * 3,
        mesh=mesh,
        compiler_params=pltpu.CompilerParams(use_tc_tiling_on_sc=False),
        scratch_types=[
            pltpu.VMEM((2 * _CH, _C), jnp.int32),
            pltpu.VMEM((2 * _CH, _C), jnp.int32),
            pltpu.VMEM((_C, 16), _f32),
            pltpu.VMEM((_C, 16), _f32),
            pltpu.VMEM((_C, _DH), _f32),
            pltpu.VMEM((_C, _DH), _f32),
            pltpu.VMEM((128, _DH), _f32),
            pltpu.VMEM((128, _DH), _f32),
            pltpu.VMEM_SHARED((_PN, _DH), _f32),
            pltpu.SemaphoreType.DMA,
            pltpu.SemaphoreType.DMA,
            pltpu.SemaphoreType.DMA,
            pltpu.SemaphoreType.DMA,
        ],
    )
    def k(g0_h, g1_h, g2_h, r0_h, c0_h, r1_h, c1_h, w1_h, r2_h, c2_h, w2_h,
          o0_h, o1_h, o2_h,
          rowbuf, colbuf, wrA, wrB, rvA, rvB, zr, bounce, acc,
          gsA, gsB, ssA, ssB):
        c = lax.axis_index("c")
        s = lax.axis_index("s")
        goff = c * _N
        nj = 2 * _CH

        def zrow(r, carry):
            for t in range(_DH // 16):
                zr[r, pl.ds(t * 16, 16)] = jnp.zeros((16,), _f32)
            return carry
        lax.fori_loop(0, 128, zrow, 0)

        gs_h = [g0_h, g1_h, g2_h]
        rows_h = [r0_h, r1_h, r2_h]
        cols_h = [c0_h, c1_h, c2_h]
        ws_h = [None, w1_h, w2_h]
        outs_h = [o0_h, o1_h, o2_h]
        bufs = [(rvA, wrA, gsA, ssA), (rvB, wrB, gsB, ssB)]
        for kk in range(3):
            for z in range(5):
                pltpu.sync_copy(zr, acc.at[pl.ds(s * _RPT + z * 128, 128)])
            plsc.subcore_barrier()
            for half in range(2):
                pltpu.sync_copy(rows_h[kk].at[2 * s + half],
                                rowbuf.at[pl.ds(half * _CH, _CH)])
                pltpu.sync_copy(cols_h[kk].at[2 * s + half],
                                colbuf.at[pl.ds(half * _CH, _CH)])

            def adj(j, carry):
                for rb in range(_C // 16):
                    rowbuf[j, pl.ds(rb * 16, 16)] = (
                        rowbuf[j, pl.ds(rb * 16, 16)] + goff)
                return carry
            lax.fori_loop(0, 2 * _CH, adj, 0)

            g_h = gs_h[kk]
            w_h = ws_h[kk]
            weighted = w_h is not None

            def issue(j, buf):
                rvb, wrb, gsb, _ = bufs[buf]
                pltpu.async_copy(g_h.at[rowbuf.at[j]], rvb, gsb)
                if weighted:
                    base = pl.multiple_of((2 * s * _CH + j) * _C, 8)
                    pltpu.async_copy(w_h.at[pl.ds(base, _C)], wrb, gsb)

            def wait_gather(buf):
                rvb, wrb, gsb, _ = bufs[buf]
                pltpu.make_async_copy(g_h.at[rowbuf.at[0]], rvb, gsb).wait()
                if weighted:
                    pltpu.make_async_copy(
                        w_h.at[pl.ds(0, _C)], wrb, gsb).wait()

            def wait_scatter(buf):
                rvb, _, _, ssb = bufs[buf]
                pltpu.make_async_copy(rvb, acc.at[colbuf.at[0]], ssb).wait()

            issue(0, 0)

            def pair(i, carry):
                for b in range(2):
                    j = 2 * i + b
                    rvb, wrb, _, ssb = bufs[b]

                    @pl.when(j + 1 < nj)
                    def _():
                        @pl.when(j >= 1)
                        def _():
                            wait_scatter(1 - b)
                        issue(j + 1, 1 - b)

                    wait_gather(b)
                    if weighted:
                        @plsc.parallel_loop(0, _C, unroll=8)
                        def scale(r):
                            wv = wrb[r, :]
                            for t in range(_DH // 16):
                                rvb[r, pl.ds(t * 16, 16)] = (
                                    rvb[r, pl.ds(t * 16, 16)] * wv)
                    pltpu.async_copy(rvb, acc.at[colbuf.at[j]], ssb,
                                     add=True)
                return carry
            lax.fori_loop(0, _CH, pair, 0)
            wait_scatter(0)
            wait_scatter(1)
            plsc.subcore_barrier()
            for z in range(5):
                pltpu.sync_copy(acc.at[pl.ds(s * _RPT + z * 128, 128)], bounce)
                pltpu.sync_copy(
                    bounce,
                    outs_h[kk].at[pl.ds(c * _PN + s * _RPT + z * 128, 128)])

    return k(g0, g1, g2, r0, c0, r1, c1, w1, r2, c2, w2)


def _dinvs(dp, kk):
    deg = 1.0 + dp[0, kk, :, 0:1] + dp[1, kk, :, 0:1]
    dinv = lax.rsqrt(deg)
    return dinv, 1.0 / deg


def _store_g(g_ref, ga):
    g_ref[0] = ga[:, :_DH]
    g_ref[1] = ga[:, _DH:]


def _tc_layer0(x, W0, degp):
    def body(x_ref, w_ref, dp_ref, hw_ref, g0_ref, g1_ref, g2_ref):
        xa = x_ref[...]
        hw = lax.dot_general(xa, w_ref[...], (((1,), (1,)), ((), ())),
                             preferred_element_type=_f32)
        hw_ref[...] = hw
        dp = dp_ref[...]
        for kk, g_ref in enumerate((g0_ref, g1_ref, g2_ref)):
            dinv, _ = _dinvs(dp, kk)
            _store_g(g_ref, dinv * hw)

    return pl.pallas_call(
        body,
        grid=(_G,),
        in_specs=[
            pl.BlockSpec((_R, _D), lambda i: (i, 0)),
            pl.BlockSpec((_D, _D), lambda i: (0, 0)),
            pl.BlockSpec((2, 3, _R, 16), lambda i: (0, 0, i, 0)),
        ],
        out_specs=[pl.BlockSpec((_R, _D), lambda i: (i, 0))]
        + [pl.BlockSpec((2, _R, _DH), lambda i: (0, i, 0))] * 3,
        out_shape=[jax.ShapeDtypeStruct((_N, _D), _f32)]
        + [jax.ShapeDtypeStruct((2, _N, _DH), _f32)] * 3,
    )(x, W0, degp)


def _tc_combine(p0, p1, p2, hw, degp, b, Wn):
    def body(p0_ref, p1_ref, p2_ref, hw_ref, dp_ref, b_ref, wn_ref,
             hwn_ref, g0_ref, g1_ref, g2_ref):
        hw_ = hw_ref[...]
        dp = dp_ref[...]
        bv = b_ref[...]
        wn = wn_ref[...]
        acc = jnp.zeros((_R, _D), _f32)
        for kk, p_ref in enumerate((p0_ref, p1_ref, p2_ref)):
            p = p_ref[...]
            seg = jnp.concatenate([p[0], p[1]], axis=1)
            dinv, dinv2 = _dinvs(dp, kk)
            sk = dinv * seg + dinv2 * hw_ + bv
            rk = jnp.maximum(sk, 0.0)
            acc = acc + lax.dot_general(
                rk, wn[:, kk * _D:(kk + 1) * _D], (((1,), (1,)), ((), ())),
                preferred_element_type=_f32)
        hwn_ref[...] = acc
        for kk, g_ref in enumerate((g0_ref, g1_ref, g2_ref)):
            dinv, _ = _dinvs(dp, kk)
            _store_g(g_ref, dinv * acc)

    return pl.pallas_call(
        body,
        grid=(_G,),
        in_specs=[
            pl.BlockSpec((2, _R, _DH), lambda i: (0, i, 0)),
            pl.BlockSpec((2, _R, _DH), lambda i: (0, i, 0)),
            pl.BlockSpec((2, _R, _DH), lambda i: (0, i, 0)),
            pl.BlockSpec((_R, _D), lambda i: (i, 0)),
            pl.BlockSpec((2, 3, _R, 16), lambda i: (0, 0, i, 0)),
            pl.BlockSpec((1, _D), lambda i: (0, 0)),
            pl.BlockSpec((_D, 3 * _D), lambda i: (0, 0)),
        ],
        out_specs=[pl.BlockSpec((_R, _D), lambda i: (i, 0))]
        + [pl.BlockSpec((2, _R, _DH), lambda i: (0, i, 0))] * 3,
        out_shape=[jax.ShapeDtypeStruct((_N, _D), _f32)]
        + [jax.ShapeDtypeStruct((2, _N, _DH), _f32)] * 3,
    )(p0, p1, p2, hw, degp, b, Wn)


def _tc_final(p0, p1, p2, hw, degp, b, Wc, bc):
    def body(p0_ref, p1_ref, p2_ref, hw_ref, dp_ref, b_ref, wc_ref, bc_ref,
             h_ref, ls_ref):
        hw_ = hw_ref[...]
        dp = dp_ref[...]
        bv = b_ref[...]
        wc = wc_ref[...]
        y = jnp.zeros((_R, _OUT), _f32) + bc_ref[...]
        for kk, p_ref in enumerate((p0_ref, p1_ref, p2_ref)):
            p = p_ref[...]
            seg = jnp.concatenate([p[0], p[1]], axis=1)
            dinv, dinv2 = _dinvs(dp, kk)
            sk = dinv * seg + dinv2 * hw_ + bv
            rk = jnp.maximum(sk, 0.0)
            h_ref[:, kk * _D:(kk + 1) * _D] = rk
            y = y + lax.dot_general(
                rk, wc[:, kk * _D:(kk + 1) * _D], (((1,), (1,)), ((), ())),
                preferred_element_type=_f32)
        m = jnp.max(y, axis=1, keepdims=True)
        z = y - m
        ls_ref[...] = z - jnp.log(jnp.sum(jnp.exp(z), axis=1, keepdims=True))

    return pl.pallas_call(
        body,
        grid=(_G,),
        in_specs=[
            pl.BlockSpec((2, _R, _DH), lambda i: (0, i, 0)),
            pl.BlockSpec((2, _R, _DH), lambda i: (0, i, 0)),
            pl.BlockSpec((2, _R, _DH), lambda i: (0, i, 0)),
            pl.BlockSpec((_R, _D), lambda i: (i, 0)),
            pl.BlockSpec((2, 3, _R, 16), lambda i: (0, 0, i, 0)),
            pl.BlockSpec((1, _D), lambda i: (0, 0)),
            pl.BlockSpec((_OUT, 3 * _D), lambda i: (0, 0)),
            pl.BlockSpec((1, _OUT), lambda i: (0, 0)),
        ],
        out_specs=[
            pl.BlockSpec((_R, 3 * _D), lambda i: (i, 0)),
            pl.BlockSpec((_R, _OUT), lambda i: (i, 0)),
        ],
        out_shape=[
            jax.ShapeDtypeStruct((_N, 3 * _D), _f32),
            jax.ShapeDtypeStruct((_N, _OUT), _f32),
        ],
    )(p0, p1, p2, hw, degp, b, Wc, bc)


def kernel(x, edge_index, edge_in, edge_out, in_w, out_w,
           W0, W1, W2, b0, b1, b2, Wc, bc):
    shp = (32, _CH, _C)
    r0 = edge_index[0].reshape(shp)
    c0 = edge_index[1].reshape(shp)
    r1 = edge_in[0].reshape(shp)
    c1 = edge_in[1].reshape(shp)
    r2 = edge_out[0].reshape(shp)
    c2 = edge_out[1].reshape(shp)
    w1 = jnp.broadcast_to(in_w[:, None], (_E, 16))
    w2 = jnp.broadcast_to(out_w[:, None], (_E, 16))

    degp = _sc_degrees(c0, c1, c2, w1, w2)
    degp = degp.reshape(_NC, 3, _PN, 16)[:, :, :_N, :]

    hw, g0, g1, g2 = _tc_layer0(x, W0, degp)
    Wn = [W1, W2]
    bs = [b0, b1, b2]
    for layer in range(3):
        p0, p1, p2 = _sc_propagate(
            g0.reshape(2 * _N, _DH), g1.reshape(2 * _N, _DH),
            g2.reshape(2 * _N, _DH), r0, c0, r1, c1, w1, r2, c2, w2)
        p0 = p0.reshape(_NC, _PN, _DH)[:, :_N, :]
        p1 = p1.reshape(_NC, _PN, _DH)[:, :_N, :]
        p2 = p2.reshape(_NC, _PN, _DH)[:, :_N, :]
        if layer < 2:
            hw, g0, g1, g2 = _tc_combine(p0, p1, p2, hw, degp, bs[layer],
                                         Wn[layer])
        else:
            h, ls = _tc_final(p0, p1, p2, hw, degp, bs[layer], Wc,
                              bc.reshape(1, _OUT))
    return (h, ls)


# speedup vs baseline: 12.1793x; 1.1320x over previous
import functools

import jax
import jax.numpy as jnp
from jax import lax
from jax.experimental import pallas as pl
from jax.experimental.pallas import tpu as pltpu
from jax.experimental.pallas import tpu_sc as plsc

_N = 10000
_PN = 10240
_E = 320000
_D = 128
_OUT = 40
_C = 125
_CH = _E // 32 // _C
_NC = 2
_NS = 16
_RPT = _PN // _NS
_R = 400
_G = _N // _R
_DH = _D // 2

_f32 = jnp.float32


def _sc_degrees(col0, col1, col2, w16_1, w16_2):
    mesh = plsc.VectorSubcoreMesh(core_axis_name="c", subcore_axis_name="s")

    @functools.partial(
        pl.kernel,
        out_type=jax.ShapeDtypeStruct((_NC * 3 * _PN, 16), _f32),
        mesh=mesh,
        compiler_params=pltpu.CompilerParams(use_tc_tiling_on_sc=False),
        scratch_types=[
            pltpu.VMEM((_CH, _C), jnp.int32),
            pltpu.VMEM((_C, 16), _f32),
            pltpu.VMEM((128, 16), _f32),
            pltpu.VMEM((128, 16), _f32),
            pltpu.VMEM_SHARED((_PN, 16), _f32),
        ],
    )
    def k(col0_h, col1_h, col2_h, w1_h, w2_h, out_h,
          colbuf, wrow, zb, bounce, acc):
        c = lax.axis_index("c")
        s = lax.axis_index("s")
        wid = c * _NS + s

        def zrow(r, carry):
            zb[r, :] = jnp.zeros((16,), _f32)
            return carry
        lax.fori_loop(0, 128, zrow, 0)

        cols_h = [col0_h, col1_h, col2_h]
        ws_h = [None, w1_h, w2_h]
        for kk in range(3):
            for z in range(5):
                pltpu.sync_copy(zb, acc.at[pl.ds(s * _RPT + z * 128, 128)])
            plsc.subcore_barrier()
            pltpu.sync_copy(cols_h[kk].at[wid], colbuf)
            if ws_h[kk] is None:
                def orow(r, carry):
                    wrow[r, :] = jnp.full((16,), 1.0, _f32)
                    return carry
                lax.fori_loop(0, _C, orow, 0)

                def chunk0(j, carry):
                    pltpu.sync_copy(wrow, acc.at[colbuf.at[j]], add=True)
                    return carry
                lax.fori_loop(0, _CH, chunk0, 0)
            else:
                w_h = ws_h[kk]

                def chunkw(j, carry):
                    base = (wid * _CH + j) * _C
                    pltpu.sync_copy(w_h.at[pl.ds(base, _C)], wrow)
                    pltpu.sync_copy(wrow, acc.at[colbuf.at[j]], add=True)
                    return carry
                lax.fori_loop(0, _CH, chunkw, 0)
            plsc.subcore_barrier()
            for z in range(5):
                pltpu.sync_copy(acc.at[pl.ds(s * _RPT + z * 128, 128)], bounce)
                pltpu.sync_copy(
                    bounce,
                    out_h.at[pl.ds((c * 3 + kk) * _PN + s * _RPT + z * 128,
                                   128)])

    return k(col0, col1, col2, w16_1, w16_2)


def _sc_propagate(g0, g1, g2, rr0, c0, rr1, c1, w1, rr2, c2, w2):
    mesh = plsc.VectorSubcoreMesh(core_axis_name="c", subcore_axis_name="s")
    nbuf = 2

    @functools.partial(
        pl.kernel,
        out_type=[jax.ShapeDtypeStruct((_NC * _PN, _DH), _f32)] * 3,
        mesh=mesh,
        compiler_params=pltpu.CompilerParams(use_tc_tiling_on_sc=False),
        scratch_types=[
            pltpu.VMEM((2 * _CH, _C), jnp.int32),
            pltpu.VMEM((2 * _CH, _C), jnp.int32),
            [pltpu.VMEM((_C, 16), _f32)] * nbuf,
            [pltpu.VMEM((_C, _DH), _f32)] * nbuf,
            pltpu.VMEM((128, _DH), _f32),
            pltpu.VMEM((128, _DH), _f32),
            pltpu.VMEM_SHARED((_PN, _DH), _f32),
            [pltpu.SemaphoreType.DMA] * nbuf,
            [pltpu.SemaphoreType.DMA] * nbuf,
        ],
    )
    def k(g0_h, g1_h, g2_h, r0_h, c0_h, r1_h, c1_h, w1_h, r2_h, c2_h, w2_h,
          o0_h, o1_h, o2_h,
          rowbuf, colbuf, wrs, rvs, zr, bounce, acc, gsems, ssems):
        c = lax.axis_index("c")
        s = lax.axis_index("s")
        nj = 2 * _CH

        def zrow(r, carry):
            for t in range(_DH // 16):
                zr[r, pl.ds(t * 16, 16)] = jnp.zeros((16,), _f32)
            return carry
        lax.fori_loop(0, 128, zrow, 0)

        gs_h = [g0_h, g1_h, g2_h]
        rows_h = [r0_h, r1_h, r2_h]
        cols_h = [c0_h, c1_h, c2_h]
        ws_h = [None, w1_h, w2_h]
        outs_h = [o0_h, o1_h, o2_h]
        for kk in range(3):
            for z in range(5):
                pltpu.sync_copy(zr, acc.at[pl.ds(s * _RPT + z * 128, 128)])
            plsc.subcore_barrier()
            for half in range(2):
                pltpu.sync_copy(rows_h[kk].at[c * 32 + 2 * s + half],
                                rowbuf.at[pl.ds(half * _CH, _CH)])
                pltpu.sync_copy(cols_h[kk].at[2 * s + half],
                                colbuf.at[pl.ds(half * _CH, _CH)])

            g_h = gs_h[kk]
            w_h = ws_h[kk]
            weighted = w_h is not None

            def issue(j, b):
                pltpu.async_copy(g_h.at[rowbuf.at[j]], rvs[b], gsems[b])
                if weighted:
                    base = (2 * s * _CH + j) * _C
                    pltpu.async_copy(w_h.at[pl.ds(base, _C)], wrs[b],
                                     gsems[b])

            def wait_gather(b):
                pltpu.make_async_copy(
                    g_h.at[rowbuf.at[0]], rvs[b], gsems[b]).wait()
                if weighted:
                    pltpu.make_async_copy(
                        w_h.at[pl.ds(0, _C)], wrs[b], gsems[b]).wait()

            def wait_scatter(b):
                pltpu.make_async_copy(
                    rvs[b], acc.at[colbuf.at[0]], ssems[b]).wait()

            for b in range(nbuf - 1):
                issue(b, b)

            def quad(i, carry):
                for b in range(nbuf):
                    j = nbuf * i + b
                    nb = (b + nbuf - 1) % nbuf

                    @pl.when(j + nbuf - 1 < nj)
                    def _():
                        @pl.when(j >= 1)
                        def _():
                            wait_scatter(nb)
                        issue(j + nbuf - 1, nb)

                    wait_gather(b)
                    if weighted:
                        @plsc.parallel_loop(0, _C, unroll=5)
                        def scale(r):
                            wv = wrs[b][r, :]
                            for t in range(_DH // 16):
                                rvs[b][r, pl.ds(t * 16, 16)] = (
                                    rvs[b][r, pl.ds(t * 16, 16)] * wv)
                    pltpu.async_copy(rvs[b], acc.at[colbuf.at[j]], ssems[b],
                                     add=True)
                return carry
            lax.fori_loop(0, nj // nbuf, quad, 0)
            for b in range(nbuf):
                wait_scatter(b)
            plsc.subcore_barrier()
            for z in range(5):
                pltpu.sync_copy(acc.at[pl.ds(s * _RPT + z * 128, 128)], bounce)
                pltpu.sync_copy(
                    bounce,
                    outs_h[kk].at[pl.ds(c * _PN + s * _RPT + z * 128, 128)])

    return k(g0, g1, g2, rr0, c0, rr1, c1, w1, rr2, c2, w2)


def _dinvs(dp, kk):
    deg = 1.0 + dp[0, kk, :, 0:1] + dp[1, kk, :, 0:1]
    dinv = lax.rsqrt(deg)
    return dinv, 1.0 / deg


def _store_g(g_ref, ga):
    g_ref[0] = ga[:, :_DH]
    g_ref[1] = ga[:, _DH:]


def _tc_layer0(x, W0, degp):
    def body(x_ref, w_ref, dp_ref, hw_ref, g0_ref, g1_ref, g2_ref):
        xa = x_ref[...]
        hw = lax.dot_general(xa, w_ref[...], (((1,), (1,)), ((), ())),
                             preferred_element_type=_f32)
        hw_ref[...] = hw
        dp = dp_ref[...]
        for kk, g_ref in enumerate((g0_ref, g1_ref, g2_ref)):
            dinv, _ = _dinvs(dp, kk)
            _store_g(g_ref, dinv * hw)

    return pl.pallas_call(
        body,
        grid=(_G,),
        in_specs=[
            pl.BlockSpec((_R, _D), lambda i: (i, 0)),
            pl.BlockSpec((_D, _D), lambda i: (0, 0)),
            pl.BlockSpec((2, 3, _R, 16), lambda i: (0, 0, i, 0)),
        ],
        out_specs=[pl.BlockSpec((_R, _D), lambda i: (i, 0))]
        + [pl.BlockSpec((2, _R, _DH), lambda i: (0, i, 0))] * 3,
        out_shape=[jax.ShapeDtypeStruct((_N, _D), _f32)]
        + [jax.ShapeDtypeStruct((2, _N, _DH), _f32)] * 3,
    )(x, W0, degp)


def _tc_combine(p0, p1, p2, hw, degp, b, Wn):
    def body(p0_ref, p1_ref, p2_ref, hw_ref, dp_ref, b_ref, wn_ref,
             hwn_ref, g0_ref, g1_ref, g2_ref):
        hw_ = hw_ref[...]
        dp = dp_ref[...]
        bv = b_ref[...]
        wn = wn_ref[...]
        acc = jnp.zeros((_R, _D), _f32)
        for kk, p_ref in enumerate((p0_ref, p1_ref, p2_ref)):
            p = p_ref[...]
            seg = jnp.concatenate([p[0], p[1]], axis=1)
            dinv, dinv2 = _dinvs(dp, kk)
            sk = dinv * seg + dinv2 * hw_ + bv
            rk = jnp.maximum(sk, 0.0)
            acc = acc + lax.dot_general(
                rk, wn[:, kk * _D:(kk + 1) * _D], (((1,), (1,)), ((), ())),
                preferred_element_type=_f32)
        hwn_ref[...] = acc
        for kk, g_ref in enumerate((g0_ref, g1_ref, g2_ref)):
            dinv, _ = _dinvs(dp, kk)
            _store_g(g_ref, dinv * acc)

    return pl.pallas_call(
        body,
        grid=(_G,),
        in_specs=[
            pl.BlockSpec((2, _R, _DH), lambda i: (0, i, 0)),
            pl.BlockSpec((2, _R, _DH), lambda i: (0, i, 0)),
            pl.BlockSpec((2, _R, _DH), lambda i: (0, i, 0)),
            pl.BlockSpec((_R, _D), lambda i: (i, 0)),
            pl.BlockSpec((2, 3, _R, 16), lambda i: (0, 0, i, 0)),
            pl.BlockSpec((1, _D), lambda i: (0, 0)),
            pl.BlockSpec((_D, 3 * _D), lambda i: (0, 0)),
        ],
        out_specs=[pl.BlockSpec((_R, _D), lambda i: (i, 0))]
        + [pl.BlockSpec((2, _R, _DH), lambda i: (0, i, 0))] * 3,
        out_shape=[jax.ShapeDtypeStruct((_N, _D), _f32)]
        + [jax.ShapeDtypeStruct((2, _N, _DH), _f32)] * 3,
    )(p0, p1, p2, hw, degp, b, Wn)


def _tc_final(p0, p1, p2, hw, degp, b, Wc, bc):
    def body(p0_ref, p1_ref, p2_ref, hw_ref, dp_ref, b_ref, wc_ref, bc_ref,
             h_ref, ls_ref):
        hw_ = hw_ref[...]
        dp = dp_ref[...]
        bv = b_ref[...]
        wc = wc_ref[...]
        y = jnp.zeros((_R, _OUT), _f32) + bc_ref[...]
        for kk, p_ref in enumerate((p0_ref, p1_ref, p2_ref)):
            p = p_ref[...]
            seg = jnp.concatenate([p[0], p[1]], axis=1)
            dinv, dinv2 = _dinvs(dp, kk)
            sk = dinv * seg + dinv2 * hw_ + bv
            rk = jnp.maximum(sk, 0.0)
            h_ref[:, kk * _D:(kk + 1) * _D] = rk
            y = y + lax.dot_general(
                rk, wc[:, kk * _D:(kk + 1) * _D], (((1,), (1,)), ((), ())),
                preferred_element_type=_f32)
        m = jnp.max(y, axis=1, keepdims=True)
        z = y - m
        ls_ref[...] = z - jnp.log(jnp.sum(jnp.exp(z), axis=1, keepdims=True))

    return pl.pallas_call(
        body,
        grid=(_G,),
        in_specs=[
            pl.BlockSpec((2, _R, _DH), lambda i: (0, i, 0)),
            pl.BlockSpec((2, _R, _DH), lambda i: (0, i, 0)),
            pl.BlockSpec((2, _R, _DH), lambda i: (0, i, 0)),
            pl.BlockSpec((_R, _D), lambda i: (i, 0)),
            pl.BlockSpec((2, 3, _R, 16), lambda i: (0, 0, i, 0)),
            pl.BlockSpec((1, _D), lambda i: (0, 0)),
            pl.BlockSpec((_OUT, 3 * _D), lambda i: (0, 0)),
            pl.BlockSpec((1, _OUT), lambda i: (0, 0)),
        ],
        out_specs=[
            pl.BlockSpec((_R, 3 * _D), lambda i: (i, 0)),
            pl.BlockSpec((_R, _OUT), lambda i: (i, 0)),
        ],
        out_shape=[
            jax.ShapeDtypeStruct((_N, 3 * _D), _f32),
            jax.ShapeDtypeStruct((_N, _OUT), _f32),
        ],
    )(p0, p1, p2, hw, degp, b, Wc, bc)


def kernel(x, edge_index, edge_in, edge_out, in_w, out_w,
           W0, W1, W2, b0, b1, b2, Wc, bc):
    shp = (32, _CH, _C)

    def _rows2(r):
        r3 = r.reshape(shp)
        return jnp.concatenate([r3, r3 + _N], axis=0)

    r0 = _rows2(edge_index[0])
    c0 = edge_index[1].reshape(shp)
    r1 = _rows2(edge_in[0])
    c1 = edge_in[1].reshape(shp)
    r2 = _rows2(edge_out[0])
    c2 = edge_out[1].reshape(shp)
    w1 = jnp.broadcast_to(in_w[:, None], (_E, 16))
    w2 = jnp.broadcast_to(out_w[:, None], (_E, 16))

    degp = _sc_degrees(c0, c1, c2, w1, w2)
    degp = degp.reshape(_NC, 3, _PN, 16)[:, :, :_N, :]

    hw, g0, g1, g2 = _tc_layer0(x, W0, degp)
    Wn = [W1, W2]
    bs = [b0, b1, b2]
    for layer in range(3):
        p0, p1, p2 = _sc_propagate(
            g0.reshape(2 * _N, _DH), g1.reshape(2 * _N, _DH),
            g2.reshape(2 * _N, _DH), r0, c0, r1, c1, w1, r2, c2, w2)
        p0 = p0.reshape(_NC, _PN, _DH)[:, :_N, :]
        p1 = p1.reshape(_NC, _PN, _DH)[:, :_N, :]
        p2 = p2.reshape(_NC, _PN, _DH)[:, :_N, :]
        if layer < 2:
            hw, g0, g1, g2 = _tc_combine(p0, p1, p2, hw, degp, bs[layer],
                                         Wn[layer])
        else:
            h, ls = _tc_final(p0, p1, p2, hw, degp, bs[layer], Wc,
                              bc.reshape(1, _OUT))
    return (h, ls)


# speedup vs baseline: 12.5764x; 1.0326x over previous
import functools

import jax
import jax.numpy as jnp
from jax import lax
from jax.experimental import pallas as pl
from jax.experimental.pallas import tpu as pltpu
from jax.experimental.pallas import tpu_sc as plsc

_N = 10000
_PN = 10240
_E = 320000
_D = 128
_OUT = 40
_C = 125
_CH = _E // 32 // _C
_NC = 2
_NS = 16
_RPT = _PN // _NS
_R = 400
_G = _N // _R
_DH = _D // 2

_f32 = jnp.float32


def _sc_degrees(col0, col1, col2, w16_1, w16_2):
    mesh = plsc.VectorSubcoreMesh(core_axis_name="c", subcore_axis_name="s")

    @functools.partial(
        pl.kernel,
        out_type=jax.ShapeDtypeStruct((_NC * 3 * _PN, 16), _f32),
        mesh=mesh,
        compiler_params=pltpu.CompilerParams(use_tc_tiling_on_sc=False),
        scratch_types=[
            pltpu.VMEM((_CH, _C), jnp.int32),
            pltpu.VMEM((_C, 16), _f32),
            pltpu.VMEM((128, 16), _f32),
            pltpu.VMEM((128, 16), _f32),
            pltpu.VMEM_SHARED((_PN, 16), _f32),
        ],
    )
    def k(col0_h, col1_h, col2_h, w1_h, w2_h, out_h,
          colbuf, wrow, zb, bounce, acc):
        c = lax.axis_index("c")
        s = lax.axis_index("s")
        wid = c * _NS + s

        def zrow(r, carry):
            zb[r, :] = jnp.zeros((16,), _f32)
            return carry
        lax.fori_loop(0, 128, zrow, 0)

        cols_h = [col0_h, col1_h, col2_h]
        ws_h = [None, w1_h, w2_h]
        for kk in range(3):
            for z in range(5):
                pltpu.sync_copy(zb, acc.at[pl.ds(s * _RPT + z * 128, 128)])
            plsc.subcore_barrier()
            pltpu.sync_copy(cols_h[kk].at[wid], colbuf)
            if ws_h[kk] is None:
                def orow(r, carry):
                    wrow[r, :] = jnp.full((16,), 1.0, _f32)
                    return carry
                lax.fori_loop(0, _C, orow, 0)

                def chunk0(j, carry):
                    pltpu.sync_copy(wrow, acc.at[colbuf.at[j]], add=True)
                    return carry
                lax.fori_loop(0, _CH, chunk0, 0)
            else:
                w_h = ws_h[kk]

                def chunkw(j, carry):
                    base = (wid * _CH + j) * _C
                    pltpu.sync_copy(w_h.at[pl.ds(base, _C)], wrow)
                    pltpu.sync_copy(wrow, acc.at[colbuf.at[j]], add=True)
                    return carry
                lax.fori_loop(0, _CH, chunkw, 0)
            plsc.subcore_barrier()
            for z in range(5):
                pltpu.sync_copy(acc.at[pl.ds(s * _RPT + z * 128, 128)], bounce)
                pltpu.sync_copy(
                    bounce,
                    out_h.at[pl.ds((c * 3 + kk) * _PN + s * _RPT + z * 128,
                                   128)])

    return k(col0, col1, col2, w16_1, w16_2)


def _sc_propagate(g0, g1, g2, rr0, c0, rr1, c1, w1, rr2, c2, w2):
    mesh = plsc.VectorSubcoreMesh(core_axis_name="c", subcore_axis_name="s")
    nbuf = 2

    @functools.partial(
        pl.kernel,
        out_type=[jax.ShapeDtypeStruct((_NC * _PN, _DH), _f32)] * 3,
        mesh=mesh,
        compiler_params=pltpu.CompilerParams(use_tc_tiling_on_sc=False),
        scratch_types=[
            pltpu.VMEM((2 * _CH, _C), jnp.int32),
            pltpu.VMEM((2 * _CH, _C), jnp.int32),
            [pltpu.VMEM((_C, 16), _f32)] * nbuf,
            [pltpu.VMEM((_C, _DH), _f32)] * nbuf,
            pltpu.VMEM((128, _DH), _f32),
            pltpu.VMEM((128, _DH), _f32),
            pltpu.VMEM_SHARED((_PN, _DH), _f32),
            [pltpu.SemaphoreType.DMA] * nbuf,
            [pltpu.SemaphoreType.DMA] * nbuf,
        ],
    )
    def k(g0_h, g1_h, g2_h, r0_h, c0_h, r1_h, c1_h, w1_h, r2_h, c2_h, w2_h,
          o0_h, o1_h, o2_h,
          rowbuf, colbuf, wrs, rvs, zr, bounce, acc, gsems, ssems):
        c = lax.axis_index("c")
        s = lax.axis_index("s")
        nj = 2 * _CH

        def zrow(r, carry):
            for t in range(_DH // 16):
                zr[r, pl.ds(t * 16, 16)] = jnp.zeros((16,), _f32)
            return carry
        lax.fori_loop(0, 128, zrow, 0)

        gs_h = [g0_h, g1_h, g2_h]
        rows_h = [r0_h, r1_h, r2_h]
        cols_h = [c0_h, c1_h, c2_h]
        ws_h = [None, w1_h, w2_h]
        outs_h = [o0_h, o1_h, o2_h]
        for kk in range(3):
            for z in range(5):
                pltpu.sync_copy(zr, acc.at[pl.ds(s * _RPT + z * 128, 128)])
            plsc.subcore_barrier()
            for half in range(2):
                pltpu.sync_copy(rows_h[kk].at[c * 32 + 2 * s + half],
                                rowbuf.at[pl.ds(half * _CH, _CH)])
                pltpu.sync_copy(cols_h[kk].at[2 * s + half],
                                colbuf.at[pl.ds(half * _CH, _CH)])

            g_h = gs_h[kk]
            w_h = ws_h[kk]
            weighted = w_h is not None

            def issue(j, b):
                pltpu.async_copy(g_h.at[rowbuf.at[j]], rvs[b], gsems[b])
                if weighted:
                    base = (2 * s * _CH + j) * _C
                    pltpu.async_copy(w_h.at[pl.ds(base, _C)], wrs[b],
                                     gsems[b])

            def wait_gather(b):
                pltpu.make_async_copy(
                    g_h.at[rowbuf.at[0]], rvs[b], gsems[b]).wait()
                if weighted:
                    pltpu.make_async_copy(
                        w_h.at[pl.ds(0, _C)], wrs[b], gsems[b]).wait()

            def wait_scatter(b):
                pltpu.make_async_copy(
                    rvs[b], acc.at[colbuf.at[0]], ssems[b]).wait()

            for b in range(nbuf - 1):
                issue(b, b)

            def quad(i, carry):
                for b in range(nbuf):
                    j = nbuf * i + b
                    nb = (b + nbuf - 1) % nbuf

                    @pl.when(j + nbuf - 1 < nj)
                    def _():
                        @pl.when(j >= 1)
                        def _():
                            wait_scatter(nb)
                        issue(j + nbuf - 1, nb)

                    wait_gather(b)
                    if weighted:
                        @plsc.parallel_loop(0, _C, unroll=5)
                        def scale(r):
                            wv = wrs[b][r, :]
                            for t in range(_DH // 16):
                                rvs[b][r, pl.ds(t * 16, 16)] = (
                                    rvs[b][r, pl.ds(t * 16, 16)] * wv)
                    pltpu.async_copy(rvs[b], acc.at[colbuf.at[j]], ssems[b],
                                     add=True)
                return carry
            lax.fori_loop(0, nj // nbuf, quad, 0)
            for b in range(nbuf):
                wait_scatter(b)
            plsc.subcore_barrier()
            for z in range(5):
                pltpu.sync_copy(acc.at[pl.ds(s * _RPT + z * 128, 128)], bounce)
                pltpu.sync_copy(
                    bounce,
                    outs_h[kk].at[pl.ds(c * _PN + s * _RPT + z * 128, 128)])

    return k(g0, g1, g2, rr0, c0, rr1, c1, w1, rr2, c2, w2)


def _dinvs(dp, kk):
    deg = 1.0 + dp[0, kk, :, 0:1] + dp[1, kk, :, 0:1]
    dinv = lax.rsqrt(deg)
    return dinv, 1.0 / deg


def _store_g(g_ref, ga):
    g_ref[0] = ga[:, :_DH]
    g_ref[1] = ga[:, _DH:]


def _tc_layer0(x, W0, degp):
    def body(x_ref, w_ref, dp_ref, hw_ref, g0_ref, g1_ref, g2_ref):
        xa = x_ref[...]
        hw = lax.dot_general(xa, w_ref[...], (((1,), (1,)), ((), ())),
                             preferred_element_type=_f32)
        hw_ref[...] = hw
        dp = dp_ref[...]
        for kk, g_ref in enumerate((g0_ref, g1_ref, g2_ref)):
            dinv, _ = _dinvs(dp, kk)
            _store_g(g_ref, dinv * hw)

    return pl.pallas_call(
        body,
        grid=(_G,),
        in_specs=[
            pl.BlockSpec((_R, _D), lambda i: (i, 0)),
            pl.BlockSpec((_D, _D), lambda i: (0, 0)),
            pl.BlockSpec((2, 3, _R, 16), lambda i: (0, 0, i, 0)),
        ],
        out_specs=[pl.BlockSpec((_R, _D), lambda i: (i, 0))]
        + [pl.BlockSpec((2, _R, _DH), lambda i: (0, i, 0))] * 3,
        out_shape=[jax.ShapeDtypeStruct((_N, _D), _f32)]
        + [jax.ShapeDtypeStruct((2, _N, _DH), _f32)] * 3,
    )(x, W0, degp)


def _tc_combine(p0, p1, p2, hw, degp, b, Wn):
    def body(p0_ref, p1_ref, p2_ref, hw_ref, dp_ref, b_ref, wn_ref,
             hwn_ref, g0_ref, g1_ref, g2_ref):
        hw_ = hw_ref[...]
        dp = dp_ref[...]
        bv = b_ref[...]
        wn = wn_ref[...]
        acc = jnp.zeros((_R, _D), _f32)
        for kk, p_ref in enumerate((p0_ref, p1_ref, p2_ref)):
            p = p_ref[...]
            seg = jnp.concatenate([p[0], p[1]], axis=1)
            dinv, dinv2 = _dinvs(dp, kk)
            sk = dinv * seg + dinv2 * hw_ + bv
            rk = jnp.maximum(sk, 0.0)
            acc = acc + lax.dot_general(
                rk, wn[:, kk * _D:(kk + 1) * _D], (((1,), (1,)), ((), ())),
                preferred_element_type=_f32)
        hwn_ref[...] = acc
        for kk, g_ref in enumerate((g0_ref, g1_ref, g2_ref)):
            dinv, _ = _dinvs(dp, kk)
            _store_g(g_ref, dinv * acc)

    return pl.pallas_call(
        body,
        grid=(_G,),
        in_specs=[
            pl.BlockSpec((2, _R, _DH), lambda i: (0, i, 0)),
            pl.BlockSpec((2, _R, _DH), lambda i: (0, i, 0)),
            pl.BlockSpec((2, _R, _DH), lambda i: (0, i, 0)),
            pl.BlockSpec((_R, _D), lambda i: (i, 0)),
            pl.BlockSpec((2, 3, _R, 16), lambda i: (0, 0, i, 0)),
            pl.BlockSpec((1, _D), lambda i: (0, 0)),
            pl.BlockSpec((_D, 3 * _D), lambda i: (0, 0)),
        ],
        out_specs=[pl.BlockSpec((_R, _D), lambda i: (i, 0))]
        + [pl.BlockSpec((2, _R, _DH), lambda i: (0, i, 0))] * 3,
        out_shape=[jax.ShapeDtypeStruct((_N, _D), _f32)]
        + [jax.ShapeDtypeStruct((2, _N, _DH), _f32)] * 3,
    )(p0, p1, p2, hw, degp, b, Wn)


def _tc_final(p0, p1, p2, hw, degp, b, Wc, bc):
    def body(p0_ref, p1_ref, p2_ref, hw_ref, dp_ref, b_ref, wc_ref, bc_ref,
             h_ref, ls_ref):
        hw_ = hw_ref[...]
        dp = dp_ref[...]
        bv = b_ref[...]
        wc = wc_ref[...]
        y = jnp.zeros((_R, _OUT), _f32) + bc_ref[...]
        for kk, p_ref in enumerate((p0_ref, p1_ref, p2_ref)):
            p = p_ref[...]
            seg = jnp.concatenate([p[0], p[1]], axis=1)
            dinv, dinv2 = _dinvs(dp, kk)
            sk = dinv * seg + dinv2 * hw_ + bv
            rk = jnp.maximum(sk, 0.0)
            h_ref[:, kk * _D:(kk + 1) * _D] = rk
            y = y + lax.dot_general(
                rk, wc[:, kk * _D:(kk + 1) * _D], (((1,), (1,)), ((), ())),
                preferred_element_type=_f32)
        m = jnp.max(y, axis=1, keepdims=True)
        z = y - m
        ls_ref[...] = z - jnp.log(jnp.sum(jnp.exp(z), axis=1, keepdims=True))

    return pl.pallas_call(
        body,
        grid=(_G,),
        in_specs=[
            pl.BlockSpec((2, _R, _DH), lambda i: (0, i, 0)),
            pl.BlockSpec((2, _R, _DH), lambda i: (0, i, 0)),
            pl.BlockSpec((2, _R, _DH), lambda i: (0, i, 0)),
            pl.BlockSpec((_R, _D), lambda i: (i, 0)),
            pl.BlockSpec((2, 3, _R, 16), lambda i: (0, 0, i, 0)),
            pl.BlockSpec((1, _D), lambda i: (0, 0)),
            pl.BlockSpec((_OUT, 3 * _D), lambda i: (0, 0)),
            pl.BlockSpec((1, _OUT), lambda i: (0, 0)),
        ],
        out_specs=[
            pl.BlockSpec((_R, 3 * _D), lambda i: (i, 0)),
            pl.BlockSpec((_R, _OUT), lambda i: (i, 0)),
        ],
        out_shape=[
            jax.ShapeDtypeStruct((_N, 3 * _D), _f32),
            jax.ShapeDtypeStruct((_N, _OUT), _f32),
        ],
    )(p0, p1, p2, hw, degp, b, Wc, bc)


def kernel(x, edge_index, edge_in, edge_out, in_w, out_w,
           W0, W1, W2, b0, b1, b2, Wc, bc):
    shp = (32, _CH, _C)

    def _rows2(r):
        r3 = r.reshape(shp)
        return jnp.concatenate([r3, r3 + _N], axis=0)

    r0 = _rows2(edge_index[0])
    c0 = edge_index[1].reshape(shp)
    r1 = _rows2(edge_in[0])
    c1 = edge_in[1].reshape(shp)
    r2 = _rows2(edge_out[0])
    c2 = edge_out[1].reshape(shp)
    w1 = jnp.broadcast_to(in_w[:, None], (_E, 16))
    w2 = jnp.broadcast_to(out_w[:, None], (_E, 16))

    degp = _sc_degrees(c0, c1, c2, w1, w2).reshape(_NC, 3, _PN, 16)

    hw, g0, g1, g2 = _tc_layer0(x, W0, degp)
    Wn = [W1, W2]
    bs = [b0, b1, b2]
    for layer in range(3):
        p0, p1, p2 = _sc_propagate(
            g0.reshape(2 * _N, _DH), g1.reshape(2 * _N, _DH),
            g2.reshape(2 * _N, _DH), r0, c0, r1, c1, w1, r2, c2, w2)
        p0 = p0.reshape(_NC, _PN, _DH)
        p1 = p1.reshape(_NC, _PN, _DH)
        p2 = p2.reshape(_NC, _PN, _DH)
        if layer < 2:
            hw, g0, g1, g2 = _tc_combine(p0, p1, p2, hw, degp, bs[layer],
                                         Wn[layer])
        else:
            h, ls = _tc_final(p0, p1, p2, hw, degp, bs[layer], Wc,
                              bc.reshape(1, _OUT))
    return (h, ls)


# speedup vs baseline: 14.0725x; 1.1190x over previous
import functools

import jax
import jax.numpy as jnp
from jax import lax
from jax.experimental import pallas as pl
from jax.experimental.pallas import tpu as pltpu
from jax.experimental.pallas import tpu_sc as plsc

_N = 10000
_PN = 10240
_E = 320000
_D = 128
_OUT = 40
_C = 125
_CH = _E // 32 // _C
_NC = 2
_NS = 16
_RPT = _PN // _NS
_R = 400
_G = _N // _R
_DH = _D // 2

_f32 = jnp.float32


def _sc_degrees(col0, col1, col2, w16_1, w16_2):
    mesh = plsc.VectorSubcoreMesh(core_axis_name="c", subcore_axis_name="s")

    @functools.partial(
        pl.kernel,
        out_type=jax.ShapeDtypeStruct((_NC * 3 * _PN, 16), _f32),
        mesh=mesh,
        compiler_params=pltpu.CompilerParams(use_tc_tiling_on_sc=False),
        scratch_types=[
            pltpu.VMEM((_CH, _C), jnp.int32),
            pltpu.VMEM((_C, 16), _f32),
            pltpu.VMEM((128, 16), _f32),
            pltpu.VMEM((128, 16), _f32),
            pltpu.VMEM_SHARED((_PN, 16), _f32),
        ],
    )
    def k(col0_h, col1_h, col2_h, w1_h, w2_h, out_h,
          colbuf, wrow, zb, bounce, acc):
        c = lax.axis_index("c")
        s = lax.axis_index("s")
        wid = c * _NS + s

        def zrow(r, carry):
            zb[r, :] = jnp.zeros((16,), _f32)
            return carry
        lax.fori_loop(0, 128, zrow, 0)

        cols_h = [col0_h, col1_h, col2_h]
        ws_h = [None, w1_h, w2_h]
        for kk in range(3):
            for z in range(5):
                pltpu.sync_copy(zb, acc.at[pl.ds(s * _RPT + z * 128, 128)])
            plsc.subcore_barrier()
            pltpu.sync_copy(cols_h[kk].at[wid], colbuf)
            if ws_h[kk] is None:
                def orow(r, carry):
                    wrow[r, :] = jnp.full((16,), 1.0, _f32)
                    return carry
                lax.fori_loop(0, _C, orow, 0)

                def chunk0(j, carry):
                    pltpu.sync_copy(wrow, acc.at[colbuf.at[j]], add=True)
                    return carry
                lax.fori_loop(0, _CH, chunk0, 0)
            else:
                w_h = ws_h[kk]

                def chunkw(j, carry):
                    base = (wid * _CH + j) * _C
                    pltpu.sync_copy(w_h.at[pl.ds(base, _C)], wrow)
                    pltpu.sync_copy(wrow, acc.at[colbuf.at[j]], add=True)
                    return carry
                lax.fori_loop(0, _CH, chunkw, 0)
            plsc.subcore_barrier()
            for z in range(5):
                pltpu.sync_copy(acc.at[pl.ds(s * _RPT + z * 128, 128)], bounce)
                pltpu.sync_copy(
                    bounce,
                    out_h.at[pl.ds((c * 3 + kk) * _PN + s * _RPT + z * 128,
                                   128)])

    return k(col0, col1, col2, w16_1, w16_2)


def _sc_propagate(g0, g1, g2, rr0, c0, rr1, c1, w1, rr2, c2, w2):
    mesh = plsc.VectorSubcoreMesh(core_axis_name="c", subcore_axis_name="s")
    nbuf = 3

    @functools.partial(
        pl.kernel,
        out_type=[jax.ShapeDtypeStruct((_NC * _PN, _DH), _f32)] * 3,
        mesh=mesh,
        compiler_params=pltpu.CompilerParams(use_tc_tiling_on_sc=False),
        scratch_types=[
            pltpu.VMEM((2 * _CH, _C), jnp.int32),
            pltpu.VMEM((2 * _CH, _C), jnp.int32),
            [pltpu.VMEM((_C, 16), _f32)] * nbuf,
            [pltpu.VMEM((_C, _DH), _f32)] * nbuf,
            pltpu.VMEM((128, _DH), _f32),
            pltpu.VMEM((128, _DH), _f32),
            pltpu.VMEM_SHARED((_PN, _DH), _f32),
            [pltpu.SemaphoreType.DMA] * nbuf,
            [pltpu.SemaphoreType.DMA] * nbuf,
        ],
    )
    def k(g0_h, g1_h, g2_h, r0_h, c0_h, r1_h, c1_h, w1_h, r2_h, c2_h, w2_h,
          o0_h, o1_h, o2_h,
          rowbuf, colbuf, wrs, rvs, zr, bounce, acc, gsems, ssems):
        c = lax.axis_index("c")
        s = lax.axis_index("s")
        nj = 2 * _CH

        def zrow(r, carry):
            for t in range(_DH // 16):
                zr[r, pl.ds(t * 16, 16)] = jnp.zeros((16,), _f32)
            return carry
        lax.fori_loop(0, 128, zrow, 0)

        gs_h = [g0_h, g1_h, g2_h]
        rows_h = [r0_h, r1_h, r2_h]
        cols_h = [c0_h, c1_h, c2_h]
        ws_h = [None, w1_h, w2_h]
        outs_h = [o0_h, o1_h, o2_h]
        for kk in range(3):
            for z in range(5):
                pltpu.sync_copy(zr, acc.at[pl.ds(s * _RPT + z * 128, 128)])
            plsc.subcore_barrier()
            for half in range(2):
                pltpu.sync_copy(rows_h[kk].at[c * 32 + 2 * s + half],
                                rowbuf.at[pl.ds(half * _CH, _CH)])
                pltpu.sync_copy(cols_h[kk].at[2 * s + half],
                                colbuf.at[pl.ds(half * _CH, _CH)])

            g_h = gs_h[kk]
            w_h = ws_h[kk]
            weighted = w_h is not None

            def issue(j, b):
                pltpu.async_copy(g_h.at[rowbuf.at[j]], rvs[b], gsems[b])
                if weighted:
                    base = (2 * s * _CH + j) * _C
                    pltpu.async_copy(w_h.at[pl.ds(base, _C)], wrs[b],
                                     gsems[b])

            def wait_gather(b):
                pltpu.make_async_copy(
                    g_h.at[rowbuf.at[0]], rvs[b], gsems[b]).wait()
                if weighted:
                    pltpu.make_async_copy(
                        w_h.at[pl.ds(0, _C)], wrs[b], gsems[b]).wait()

            def wait_scatter(b):
                pltpu.make_async_copy(
                    rvs[b], acc.at[colbuf.at[0]], ssems[b]).wait()

            for b in range(nbuf - 1):
                issue(b, b)

            def process(j, b):
                wait_gather(b)
                if weighted:
                    @plsc.parallel_loop(0, _C, unroll=5)
                    def scale(r):
                        wv = wrs[b][r, :]
                        for t in range(_DH // 16):
                            rvs[b][r, pl.ds(t * 16, 16)] = (
                                rvs[b][r, pl.ds(t * 16, 16)] * wv)
                pltpu.async_copy(rvs[b], acc.at[colbuf.at[j]], ssems[b],
                                 add=True)

            def group(i, carry):
                for b in range(nbuf):
                    j = nbuf * i + b
                    nb = (b + nbuf - 1) % nbuf

                    @pl.when(j + nbuf - 1 < nj)
                    def _():
                        @pl.when(j >= 1)
                        def _():
                            wait_scatter(nb)
                        issue(j + nbuf - 1, nb)

                    process(j, b)
                return carry
            lax.fori_loop(0, nj // nbuf, group, 0)
            for t in range(nj % nbuf):
                j = nbuf * (nj // nbuf) + t
                process(j, j % nbuf)
            for b in range(nbuf):
                wait_scatter(b)
            plsc.subcore_barrier()
            for z in range(5):
                pltpu.sync_copy(acc.at[pl.ds(s * _RPT + z * 128, 128)], bounce)
                pltpu.sync_copy(
                    bounce,
                    outs_h[kk].at[pl.ds(c * _PN + s * _RPT + z * 128, 128)])

    return k(g0, g1, g2, rr0, c0, rr1, c1, w1, rr2, c2, w2)


def _dinvs(dp, kk):
    deg = 1.0 + dp[0, kk, :, 0:1] + dp[1, kk, :, 0:1]
    dinv = lax.rsqrt(deg)
    return dinv, 1.0 / deg


def _store_g(g_ref, ga):
    g_ref[0] = ga[:, :_DH]
    g_ref[1] = ga[:, _DH:]


def _tc_layer0(x, W0, degp):
    def body(x_ref, w_ref, dp_ref, hw_ref, g0_ref, g1_ref, g2_ref):
        xa = x_ref[...]
        hw = lax.dot_general(xa, w_ref[...], (((1,), (1,)), ((), ())),
                             preferred_element_type=_f32)
        hw_ref[...] = hw
        dp = dp_ref[...]
        for kk, g_ref in enumerate((g0_ref, g1_ref, g2_ref)):
            dinv, _ = _dinvs(dp, kk)
            _store_g(g_ref, dinv * hw)

    return pl.pallas_call(
        body,
        grid=(_G,),
        in_specs=[
            pl.BlockSpec((_R, _D), lambda i: (i, 0)),
            pl.BlockSpec((_D, _D), lambda i: (0, 0)),
            pl.BlockSpec((2, 3, _R, 16), lambda i: (0, 0, i, 0)),
        ],
        out_specs=[pl.BlockSpec((_R, _D), lambda i: (i, 0))]
        + [pl.BlockSpec((2, _R, _DH), lambda i: (0, i, 0))] * 3,
        out_shape=[jax.ShapeDtypeStruct((_N, _D), _f32)]
        + [jax.ShapeDtypeStruct((2, _N, _DH), _f32)] * 3,
    )(x, W0, degp)


def _tc_combine(p0, p1, p2, hw, degp, b, Wn):
    def body(p0_ref, p1_ref, p2_ref, hw_ref, dp_ref, b_ref, wn_ref,
             hwn_ref, g0_ref, g1_ref, g2_ref):
        hw_ = hw_ref[...]
        dp = dp_ref[...]
        bv = b_ref[...]
        wn = wn_ref[...]
        acc = jnp.zeros((_R, _D), _f32)
        for kk, p_ref in enumerate((p0_ref, p1_ref, p2_ref)):
            p = p_ref[...]
            seg = jnp.concatenate([p[0], p[1]], axis=1)
            dinv, dinv2 = _dinvs(dp, kk)
            sk = dinv * seg + dinv2 * hw_ + bv
            rk = jnp.maximum(sk, 0.0)
            acc = acc + lax.dot_general(
                rk, wn[:, kk * _D:(kk + 1) * _D], (((1,), (1,)), ((), ())),
                preferred_element_type=_f32)
        hwn_ref[...] = acc
        for kk, g_ref in enumerate((g0_ref, g1_ref, g2_ref)):
            dinv, _ = _dinvs(dp, kk)
            _store_g(g_ref, dinv * acc)

    return pl.pallas_call(
        body,
        grid=(_G,),
        in_specs=[
            pl.BlockSpec((2, _R, _DH), lambda i: (0, i, 0)),
            pl.BlockSpec((2, _R, _DH), lambda i: (0, i, 0)),
            pl.BlockSpec((2, _R, _DH), lambda i: (0, i, 0)),
            pl.BlockSpec((_R, _D), lambda i: (i, 0)),
            pl.BlockSpec((2, 3, _R, 16), lambda i: (0, 0, i, 0)),
            pl.BlockSpec((1, _D), lambda i: (0, 0)),
            pl.BlockSpec((_D, 3 * _D), lambda i: (0, 0)),
        ],
        out_specs=[pl.BlockSpec((_R, _D), lambda i: (i, 0))]
        + [pl.BlockSpec((2, _R, _DH), lambda i: (0, i, 0))] * 3,
        out_shape=[jax.ShapeDtypeStruct((_N, _D), _f32)]
        + [jax.ShapeDtypeStruct((2, _N, _DH), _f32)] * 3,
    )(p0, p1, p2, hw, degp, b, Wn)


def _tc_final(p0, p1, p2, hw, degp, b, Wc, bc):
    def body(p0_ref, p1_ref, p2_ref, hw_ref, dp_ref, b_ref, wc_ref, bc_ref,
             h_ref, ls_ref):
        hw_ = hw_ref[...]
        dp = dp_ref[...]
        bv = b_ref[...]
        wc = wc_ref[...]
        y = jnp.zeros((_R, _OUT), _f32) + bc_ref[...]
        for kk, p_ref in enumerate((p0_ref, p1_ref, p2_ref)):
            p = p_ref[...]
            seg = jnp.concatenate([p[0], p[1]], axis=1)
            dinv, dinv2 = _dinvs(dp, kk)
            sk = dinv * seg + dinv2 * hw_ + bv
            rk = jnp.maximum(sk, 0.0)
            h_ref[:, kk * _D:(kk + 1) * _D] = rk
            y = y + lax.dot_general(
                rk, wc[:, kk * _D:(kk + 1) * _D], (((1,), (1,)), ((), ())),
                preferred_element_type=_f32)
        m = jnp.max(y, axis=1, keepdims=True)
        z = y - m
        ls_ref[...] = z - jnp.log(jnp.sum(jnp.exp(z), axis=1, keepdims=True))

    return pl.pallas_call(
        body,
        grid=(_G,),
        in_specs=[
            pl.BlockSpec((2, _R, _DH), lambda i: (0, i, 0)),
            pl.BlockSpec((2, _R, _DH), lambda i: (0, i, 0)),
            pl.BlockSpec((2, _R, _DH), lambda i: (0, i, 0)),
            pl.BlockSpec((_R, _D), lambda i: (i, 0)),
            pl.BlockSpec((2, 3, _R, 16), lambda i: (0, 0, i, 0)),
            pl.BlockSpec((1, _D), lambda i: (0, 0)),
            pl.BlockSpec((_OUT, 3 * _D), lambda i: (0, 0)),
            pl.BlockSpec((1, _OUT), lambda i: (0, 0)),
        ],
        out_specs=[
            pl.BlockSpec((_R, 3 * _D), lambda i: (i, 0)),
            pl.BlockSpec((_R, _OUT), lambda i: (i, 0)),
        ],
        out_shape=[
            jax.ShapeDtypeStruct((_N, 3 * _D), _f32),
            jax.ShapeDtypeStruct((_N, _OUT), _f32),
        ],
    )(p0, p1, p2, hw, degp, b, Wc, bc)


def kernel(x, edge_index, edge_in, edge_out, in_w, out_w,
           W0, W1, W2, b0, b1, b2, Wc, bc):
    shp = (32, _CH, _C)

    def _rows2(r):
        r3 = r.reshape(shp)
        return jnp.concatenate([r3, r3 + _N], axis=0)

    r0 = _rows2(edge_index[0])
    c0 = edge_index[1].reshape(shp)
    r1 = _rows2(edge_in[0])
    c1 = edge_in[1].reshape(shp)
    r2 = _rows2(edge_out[0])
    c2 = edge_out[1].reshape(shp)
    w1 = jnp.broadcast_to(in_w[:, None], (_E, 16))
    w2 = jnp.broadcast_to(out_w[:, None], (_E, 16))

    degp = _sc_degrees(c0, c1, c2, w1, w2).reshape(_NC, 3, _PN, 16)

    hw, g0, g1, g2 = _tc_layer0(x, W0, degp)
    Wn = [W1, W2]
    bs = [b0, b1, b2]
    for layer in range(3):
        p0, p1, p2 = _sc_propagate(
            g0.reshape(2 * _N, _DH), g1.reshape(2 * _N, _DH),
            g2.reshape(2 * _N, _DH), r0, c0, r1, c1, w1, r2, c2, w2)
        p0 = p0.reshape(_NC, _PN, _DH)
        p1 = p1.reshape(_NC, _PN, _DH)
        p2 = p2.reshape(_NC, _PN, _DH)
        if layer < 2:
            hw, g0, g1, g2 = _tc_combine(p0, p1, p2, hw, degp, bs[layer],
                                         Wn[layer])
        else:
            h, ls = _tc_final(p0, p1, p2, hw, degp, bs[layer], Wc,
                              bc.reshape(1, _OUT))
    return (h, ls)
